# Initial kernel scaffold; baseline (speedup 1.0000x reference)
import jax
import jax.numpy as jnp
from jax.experimental import pallas as pl


def kernel(x, edge_index, W1, b1, gamma, beta, W2, b2):
    raise NotImplementedError("write your pallas kernel here")


# speedup vs baseline: 15.3712x; 15.3712x over previous
import functools

import jax
import jax.numpy as jnp
from jax import lax
from jax.experimental import pallas as pl
from jax.experimental.pallas import tpu as pltpu
from jax.experimental.pallas import tpu_sc as plsc

N = 50000
E = 800000
IN = 64
H = 128

NC = 2
NS = 16
K = 128
CW = 32
NCHUNK = H // CW

EE = E + N
NB_AG = 416
PIECE = 26
NPIECE = NB_AG // PIECE
EPAD = NS * NB_AG * K
NB_DG = 208
DUMP = N
NROW = N + 8
ZP = 112

STRIPE = 3136
STRIPE_LAST = N - 15 * STRIPE

_mesh = plsc.VectorSubcoreMesh(
    core_axis_name="c", subcore_axis_name="s", num_cores=NC, num_subcores=NS)

_sc_params = pltpu.CompilerParams(use_tc_tiling_on_sc=False)

f32 = jnp.float32


def _fill(ref, n, value):
    v = jnp.full((16,), value, dtype=f32)

    def body(i, _):
        ref[pl.ds(i * 16, 16)] = v
        return 0

    lax.fori_loop(0, n // 16, body, 0)


def _stripe(sid):
    return STRIPE * sid


@functools.partial(
    pl.kernel,
    out_type=jax.ShapeDtypeStruct((NC * N,), f32),
    mesh=_mesh,
    compiler_params=_sc_params,
    scratch_types=[
        pltpu.VMEM_SHARED((NROW,), f32),
        pltpu.VMEM((NB_DG, K), jnp.int32),
        pltpu.VMEM((K,), f32),
        pltpu.VMEM((STRIPE,), f32),
    ],
)
def _sc_deg(dst_hbm, out_hbm, deg_sp, idxb, onesv, zbuf):
    cid = lax.axis_index("c")
    sid = lax.axis_index("s")
    _fill(onesv, K, 1.0)
    _fill(zbuf, STRIPE, 0.0)
    off = _stripe(sid)

    @pl.when(sid < NS - 1)
    def _():
        pltpu.sync_copy(zbuf, deg_sp.at[pl.ds(off, STRIPE)])

    @pl.when(sid == NS - 1)
    def _():
        pltpu.sync_copy(zbuf.at[pl.ds(0, STRIPE_LAST)],
                        deg_sp.at[pl.ds(off, STRIPE_LAST)])
        pltpu.sync_copy(zbuf.at[pl.ds(0, 8)], deg_sp.at[pl.ds(N, 8)])

    plsc.subcore_barrier()

    wid = cid * NS + sid
    pltpu.sync_copy(dst_hbm.at[wid], idxb)

    def body(i, _):
        pltpu.sync_copy(onesv, deg_sp.at[idxb.at[i]], add=True)
        return 0

    lax.fori_loop(0, NB_DG, body, 0)
    plsc.subcore_barrier()

    @pl.when(sid < NS - 1)
    def _():
        pltpu.sync_copy(deg_sp.at[pl.ds(off, STRIPE)], zbuf)
        pltpu.sync_copy(zbuf, out_hbm.at[pl.ds(cid * N + off, STRIPE)])

    @pl.when(sid == NS - 1)
    def _():
        pltpu.sync_copy(deg_sp.at[pl.ds(off, STRIPE_LAST)],
                        zbuf.at[pl.ds(0, STRIPE_LAST)])
        pltpu.sync_copy(zbuf.at[pl.ds(0, STRIPE_LAST)],
                        out_hbm.at[pl.ds(cid * N + off, STRIPE_LAST)])


@functools.partial(
    pl.kernel,
    out_type=[jax.ShapeDtypeStruct((N, CW), f32) for _ in range(NCHUNK)],
    mesh=_mesh,
    compiler_params=_sc_params,
    scratch_types=[
        pltpu.VMEM_SHARED((NROW, CW), f32),
        pltpu.VMEM((2, PIECE, K), jnp.int32),
        pltpu.VMEM((2, PIECE, K), jnp.int32),
        pltpu.VMEM((K, CW), f32),
        pltpu.VMEM((K, CW), f32),
        pltpu.VMEM((ZP, CW), f32),
        pltpu.SemaphoreType.DMA,
        pltpu.SemaphoreType.DMA,
        pltpu.SemaphoreType.DMA,
    ],
)
def _sc_agg(hs0, hs1, hs2, hs3, src_hbm, dst_hbm,
            o0, o1, o2, o3,
            agg_sp, srcv, dstv, rows0, rows1, zbuf, sem0, sem1, semi):
    cid = lax.axis_index("c")
    sid = lax.axis_index("s")
    off = _stripe(sid)

    def zfill(i, _):
        zbuf[pl.ds(i * 16, 16), :] = jnp.zeros((16, CW), dtype=f32)
        return 0

    def _stripe_copy(src_at, dst_at, nrows):
        def piece(p, _):
            pltpu.sync_copy(src_at(ZP * p, ZP), dst_at(ZP * p, ZP))
            return 0
        lax.fori_loop(0, nrows // ZP, piece, 0)
        rem = nrows - (nrows // ZP) * ZP
        if rem:
            pltpu.sync_copy(src_at(nrows - rem, rem), dst_at(nrows - rem, rem))

    def process(hs_j, out_j):
        lax.fori_loop(0, ZP // 16, zfill, 0)

        @pl.when(sid < NS - 1)
        def _():
            _stripe_copy(lambda o_, n: zbuf.at[pl.ds(0, n)],
                         lambda o_, n: agg_sp.at[pl.ds(off + o_, n)], STRIPE)

        @pl.when(sid == NS - 1)
        def _():
            _stripe_copy(lambda o_, n: zbuf.at[pl.ds(0, n)],
                         lambda o_, n: agg_sp.at[pl.ds(off + o_, n)],
                         STRIPE_LAST)
            pltpu.sync_copy(zbuf.at[pl.ds(0, 8)], agg_sp.at[pl.ds(N, 8)])

        plsc.subcore_barrier()

        pltpu.sync_copy(src_hbm.at[sid, pl.ds(0, PIECE)], srcv.at[0])
        pltpu.sync_copy(dst_hbm.at[sid, pl.ds(0, PIECE)], dstv.at[0])
        pltpu.async_copy(hs_j.at[srcv.at[0, 0]], rows0, sem0)

        def piece(p, _):
            sp = p % 2
            sq = 1 - sp

            @pl.when(p < NPIECE - 1)
            def _():
                nxt = pl.ds((p + 1) * PIECE, PIECE)
                pltpu.async_copy(src_hbm.at[sid, nxt], srcv.at[sq], semi)
                pltpu.async_copy(dst_hbm.at[sid, nxt], dstv.at[sq], semi)

            def pair(i, _):
                b = 2 * i
                pltpu.async_copy(hs_j.at[srcv.at[sp, b + 1]], rows1, sem1)
                pltpu.make_async_copy(
                    hs_j.at[srcv.at[sp, b]], rows0, sem0).wait()
                pltpu.sync_copy(rows0, agg_sp.at[dstv.at[sp, b]], add=True)

                @pl.when(i < PIECE // 2 - 1)
                def _():
                    pltpu.async_copy(hs_j.at[srcv.at[sp, b + 2]], rows0, sem0)

                @pl.when(i == PIECE // 2 - 1)
                def _():
                    @pl.when(p < NPIECE - 1)
                    def _():
                        nxt = pl.ds((p + 1) * PIECE, PIECE)
                        pltpu.make_async_copy(
                            src_hbm.at[sid, nxt], srcv.at[sq], semi).wait()
                        pltpu.make_async_copy(
                            dst_hbm.at[sid, nxt], dstv.at[sq], semi).wait()
                        pltpu.async_copy(
                            hs_j.at[srcv.at[sq, 0]], rows0, sem0)

                pltpu.make_async_copy(
                    hs_j.at[srcv.at[sp, b + 1]], rows1, sem1).wait()
                pltpu.sync_copy(rows1, agg_sp.at[dstv.at[sp, b + 1]], add=True)
                return 0

            lax.fori_loop(0, PIECE // 2, pair, 0)
            return 0

        lax.fori_loop(0, NPIECE, piece, 0)
        plsc.subcore_barrier()

        @pl.when(sid < NS - 1)
        def _():
            def w(p_, _):
                pltpu.sync_copy(agg_sp.at[pl.ds(off + ZP * p_, ZP)], zbuf)
                pltpu.sync_copy(zbuf, out_j.at[pl.ds(off + ZP * p_, ZP)])
                return 0
            lax.fori_loop(0, STRIPE // ZP, w, 0)

        @pl.when(sid == NS - 1)
        def _():
            def w(p_, _):
                pltpu.sync_copy(agg_sp.at[pl.ds(off + ZP * p_, ZP)], zbuf)
                pltpu.sync_copy(zbuf, out_j.at[pl.ds(off + ZP * p_, ZP)])
                return 0
            lax.fori_loop(0, STRIPE_LAST // ZP, w, 0)
            rem = STRIPE_LAST - (STRIPE_LAST // ZP) * ZP
            pltpu.sync_copy(agg_sp.at[pl.ds(off + STRIPE_LAST - rem, rem)],
                            zbuf.at[pl.ds(0, rem)])
            pltpu.sync_copy(zbuf.at[pl.ds(0, rem)],
                            out_j.at[pl.ds(off + STRIPE_LAST - rem, rem)])

        plsc.subcore_barrier()

    @pl.when(cid == 0)
    def _():
        process(hs0, o0)
        process(hs1, o1)

    @pl.when(cid == 1)
    def _():
        process(hs2, o2)
        process(hs3, o3)


RB = 2000
NRB = N // RB
EPS = 1e-5


def _deg_scale(degp_ref):
    deg = jnp.sum(degp_ref[...], axis=1, keepdims=True)
    return lax.rsqrt(deg)


def _tca_body(x_ref, w_ref, degp_ref, o0, o1, o2, o3):
    d = _deg_scale(degp_ref)
    h = jnp.dot(x_ref[...], w_ref[...], preferred_element_type=f32) * d
    for j, o in enumerate((o0, o1, o2, o3)):
        o[...] = h[:, CW * j:CW * (j + 1)]


def _tc_a(x, W1, degp):
    return pl.pallas_call(
        _tca_body,
        grid=(NRB,),
        in_specs=[
            pl.BlockSpec((RB, IN), lambda i: (i, 0)),
            pl.BlockSpec((IN, H), lambda i: (0, 0)),
            pl.BlockSpec((RB, 2), lambda i: (i, 0)),
        ],
        out_specs=[pl.BlockSpec((RB, CW), lambda i: (i, 0))] * NCHUNK,
        out_shape=[jax.ShapeDtypeStruct((N, CW), f32)] * NCHUNK,
    )(x, W1, degp)


def _tcb1_body(a0, a1, a2, a3, degp_ref, b1_ref, stats_ref):
    d = _deg_scale(degp_ref)
    for j, a in enumerate((a0, a1, a2, a3)):
        r = jnp.maximum(a[...] * d + b1_ref[pl.ds(CW * j, CW)], 0.0)
        stats_ref[0, 0, j, :] = jnp.sum(r, axis=0)
        stats_ref[0, 1, j, :] = jnp.sum(r * r, axis=0)


def _tc_b1(aggs, degp, b1):
    return pl.pallas_call(
        _tcb1_body,
        grid=(NRB,),
        in_specs=[pl.BlockSpec((RB, CW), lambda i: (i, 0))] * NCHUNK + [
            pl.BlockSpec((RB, 2), lambda i: (i, 0)),
            pl.BlockSpec((H,), lambda i: (0,)),
        ],
        out_specs=pl.BlockSpec((1, 2, NCHUNK, CW), lambda i: (i, 0, 0, 0)),
        out_shape=jax.ShapeDtypeStruct((NRB, 2, NCHUNK, CW), f32),
    )(*aggs, degp, b1)


def _tcb2_body(a0, a1, a2, a3, degp_ref, b1_ref, stats_ref, g_ref, be_ref,
               w2_ref, o0, o1, o2, o3):
    d = _deg_scale(degp_ref)
    s = stats_ref[...]
    mean = jnp.sum(s[:, 0], axis=0) * (1.0 / N)
    var = jnp.sum(s[:, 1], axis=0) * (1.0 / N) - mean * mean
    inv = lax.rsqrt(var + EPS)
    acc = jnp.zeros((RB, H), dtype=f32)
    for j, a in enumerate((a0, a1, a2, a3)):
        sl = pl.ds(CW * j, CW)
        r = jnp.maximum(a[...] * d + b1_ref[sl], 0.0)
        y = (r - mean[j]) * (inv[j] * g_ref[sl]) + be_ref[sl]
        acc = acc + jnp.dot(y, w2_ref[sl, :], preferred_element_type=f32)
    acc = acc * d
    for j, o in enumerate((o0, o1, o2, o3)):
        o[...] = acc[:, CW * j:CW * (j + 1)]


def _tc_b2(aggs, degp, b1, stats, gamma, beta, W2):
    return pl.pallas_call(
        _tcb2_body,
        grid=(NRB,),
        in_specs=[pl.BlockSpec((RB, CW), lambda i: (i, 0))] * NCHUNK + [
            pl.BlockSpec((RB, 2), lambda i: (i, 0)),
            pl.BlockSpec((H,), lambda i: (0,)),
            pl.BlockSpec((NRB, 2, NCHUNK, CW), lambda i: (0, 0, 0, 0)),
            pl.BlockSpec((H,), lambda i: (0,)),
            pl.BlockSpec((H,), lambda i: (0,)),
            pl.BlockSpec((H, H), lambda i: (0, 0)),
        ],
        out_specs=[pl.BlockSpec((RB, CW), lambda i: (i, 0))] * NCHUNK,
        out_shape=[jax.ShapeDtypeStruct((N, CW), f32)] * NCHUNK,
    )(*aggs, degp, b1, stats, gamma, beta, W2)


def _tcc_body(a0, a1, a2, a3, degp_ref, b2_ref, out_ref):
    d = _deg_scale(degp_ref)
    cols = []
    for j, a in enumerate((a0, a1, a2, a3)):
        cols.append(jnp.maximum(a[...] * d + b2_ref[pl.ds(CW * j, CW)], 0.0))
    out_ref[...] = jnp.concatenate(cols, axis=1)


def _tc_c(aggs, degp, b2):
    return pl.pallas_call(
        _tcc_body,
        grid=(NRB,),
        in_specs=[pl.BlockSpec((RB, CW), lambda i: (i, 0))] * NCHUNK + [
            pl.BlockSpec((RB, 2), lambda i: (i, 0)),
            pl.BlockSpec((H,), lambda i: (0,)),
        ],
        out_specs=pl.BlockSpec((RB, H), lambda i: (i, 0)),
        out_shape=jax.ShapeDtypeStruct((N, H), f32),
    )(*aggs, degp, b2)


def kernel(x, edge_index, W1, b1, gamma, beta, W2, b2):
    src = edge_index[0]
    dst = edge_index[1]
    loop = jnp.arange(N, dtype=jnp.int32)
    npad = EPAD - EE
    src_p = jnp.concatenate([src, loop, jnp.zeros((npad,), jnp.int32)])
    dst_p = jnp.concatenate([dst, loop, jnp.full((npad,), DUMP, jnp.int32)])
    src_ag = src_p.reshape(NS, NB_AG, K)
    dst_ag = dst_p.reshape(NS, NB_AG, K)
    dst_dg = dst_p.reshape(NC * NS, NB_DG, K)

    degp = _sc_deg(dst_dg).reshape(NC, N).T
    hs1 = _tc_a(x, W1, degp)
    agg1 = _sc_agg(*hs1, src_ag, dst_ag)
    stats = _tc_b1(agg1, degp, b1)
    hs2 = _tc_b2(agg1, degp, b1, stats, gamma, beta, W2)
    agg2 = _sc_agg(*hs2, src_ag, dst_ag)
    return _tc_c(agg2, degp, b2)


# speedup vs baseline: 24.1048x; 1.5682x over previous
import functools

import jax
import jax.numpy as jnp
from jax import lax
from jax.experimental import pallas as pl
from jax.experimental.pallas import tpu as pltpu
from jax.experimental.pallas import tpu_sc as plsc

N = 50000
E = 800000
IN = 64
H = 128

NC = 2
NS = 16
K = 128
CW = 32

EE = E + N
NB_AG = 416
PIECE = 16
NPIECE = NB_AG // PIECE
EPAD = NS * NB_AG * K
NB_DG = 208
DUMP = N
NROW = N + 8

STRIPE = 3136
STRIPE_LAST = N - 15 * STRIPE
ZP = 112

_mesh = plsc.VectorSubcoreMesh(
    core_axis_name="c", subcore_axis_name="s", num_cores=NC, num_subcores=NS)

_sc_params = pltpu.CompilerParams(use_tc_tiling_on_sc=False)

f32 = jnp.float32


def _fill(ref, n, value):
    v = jnp.full((16,), value, dtype=f32)

    def body(i, _):
        ref[pl.ds(i * 16, 16)] = v
        return 0

    lax.fori_loop(0, n // 16, body, 0)


def _stripe(sid):
    return STRIPE * sid


@functools.partial(
    pl.kernel,
    out_type=jax.ShapeDtypeStruct((NC * N,), f32),
    mesh=_mesh,
    compiler_params=_sc_params,
    scratch_types=[
        pltpu.VMEM_SHARED((NROW,), f32),
        pltpu.VMEM((NB_DG, K), jnp.int32),
        pltpu.VMEM((K,), f32),
        pltpu.VMEM((STRIPE,), f32),
    ],
)
def _sc_deg(dst_hbm, out_hbm, deg_sp, idxb, onesv, zbuf):
    cid = lax.axis_index("c")
    sid = lax.axis_index("s")
    _fill(onesv, K, 1.0)
    _fill(zbuf, STRIPE, 0.0)
    off = _stripe(sid)

    @pl.when(sid < NS - 1)
    def _():
        pltpu.sync_copy(zbuf, deg_sp.at[pl.ds(off, STRIPE)])

    @pl.when(sid == NS - 1)
    def _():
        pltpu.sync_copy(zbuf.at[pl.ds(0, STRIPE_LAST)],
                        deg_sp.at[pl.ds(off, STRIPE_LAST)])
        pltpu.sync_copy(zbuf.at[pl.ds(0, 8)], deg_sp.at[pl.ds(N, 8)])

    plsc.subcore_barrier()

    wid = cid * NS + sid
    pltpu.sync_copy(dst_hbm.at[wid], idxb)

    def body(i, _):
        pltpu.sync_copy(onesv, deg_sp.at[idxb.at[i]], add=True)
        return 0

    lax.fori_loop(0, NB_DG, body, 0)
    plsc.subcore_barrier()

    @pl.when(sid < NS - 1)
    def _():
        pltpu.sync_copy(deg_sp.at[pl.ds(off, STRIPE)], zbuf)
        pltpu.sync_copy(zbuf, out_hbm.at[pl.ds(cid * N + off, STRIPE)])

    @pl.when(sid == NS - 1)
    def _():
        pltpu.sync_copy(deg_sp.at[pl.ds(off, STRIPE_LAST)],
                        zbuf.at[pl.ds(0, STRIPE_LAST)])
        pltpu.sync_copy(zbuf.at[pl.ds(0, STRIPE_LAST)],
                        out_hbm.at[pl.ds(cid * N + off, STRIPE_LAST)])


def _agg_impl(hs_list, src_hbm, dst_hbm, out_list,
              agg_sp, srcv, dstv, rows, zbuf, sg, ss, semi):
    cid = lax.axis_index("c")
    sid = lax.axis_index("s")
    off = _stripe(sid)

    def zfill(i, _):
        zbuf[pl.ds(i * 16, 16), :] = jnp.zeros((16, CW), dtype=f32)
        return 0

    def process(hs_j, out_j):
        lax.fori_loop(0, ZP // 16, zfill, 0)

        @pl.when(sid < NS - 1)
        def _():
            def z(p, _):
                pltpu.sync_copy(zbuf, agg_sp.at[pl.ds(off + ZP * p, ZP)])
                return 0
            lax.fori_loop(0, STRIPE // ZP, z, 0)

        @pl.when(sid == NS - 1)
        def _():
            def z(p, _):
                pltpu.sync_copy(zbuf, agg_sp.at[pl.ds(off + ZP * p, ZP)])
                return 0
            lax.fori_loop(0, STRIPE_LAST // ZP, z, 0)
            rem = STRIPE_LAST - (STRIPE_LAST // ZP) * ZP
            pltpu.sync_copy(zbuf.at[pl.ds(0, rem)],
                            agg_sp.at[pl.ds(off + STRIPE_LAST - rem, rem)])
            pltpu.sync_copy(zbuf.at[pl.ds(0, 8)], agg_sp.at[pl.ds(N, 8)])

        plsc.subcore_barrier()

        pltpu.sync_copy(src_hbm.at[sid, pl.ds(0, PIECE)], srcv.at[0])
        pltpu.sync_copy(dst_hbm.at[sid, pl.ds(0, PIECE)], dstv.at[0])
        pltpu.async_copy(hs_j.at[srcv.at[0, 0]], rows.at[0], sg.at[0])
        pltpu.async_copy(hs_j.at[srcv.at[0, 1]], rows.at[1], sg.at[1])

        def group(g, _):
            for u in range(4):
                b = 4 * g + u
                u2 = (u + 2) % 4
                if u == 2:
                    q = lax.div(g, 4)
                    qn = q + 1
                    sl = lax.rem(qn, 2)

                    @pl.when((lax.rem(g, 4) == 0) & (qn < NPIECE))
                    def _():
                        nxt = pl.ds(qn * PIECE, PIECE)
                        pltpu.async_copy(src_hbm.at[sid, nxt],
                                         srcv.at[sl], semi)
                        pltpu.async_copy(dst_hbm.at[sid, nxt],
                                         dstv.at[sl], semi)

                    @pl.when((lax.rem(g, 4) == 3) & (qn < NPIECE))
                    def _():
                        dummy = pl.ds(0, PIECE)
                        pltpu.make_async_copy(src_hbm.at[sid, dummy],
                                              srcv.at[0], semi).wait()
                        pltpu.make_async_copy(dst_hbm.at[sid, dummy],
                                              dstv.at[0], semi).wait()

                @pl.when(b >= 2)
                def _():
                    pltpu.make_async_copy(
                        rows.at[u2], agg_sp.at[dstv.at[0, 0]],
                        ss.at[u2]).wait()

                @pl.when(b + 2 < NB_AG)
                def _():
                    bn = b + 2
                    pn = lax.div(bn, PIECE)
                    pltpu.async_copy(
                        hs_j.at[srcv.at[lax.rem(pn, 2), lax.rem(bn, PIECE)]],
                        rows.at[u2], sg.at[u2])

                pltpu.make_async_copy(
                    hs_j.at[srcv.at[0, 0]], rows.at[u], sg.at[u]).wait()
                p = lax.div(b, PIECE)
                pltpu.async_copy(
                    rows.at[u], agg_sp.at[dstv.at[lax.rem(p, 2),
                                                  lax.rem(b, PIECE)]],
                    ss.at[u], add=True)
            return 0

        lax.fori_loop(0, NB_AG // 4, group, 0)
        for u in (2, 3):
            pltpu.make_async_copy(
                rows.at[u], agg_sp.at[dstv.at[0, 0]], ss.at[u]).wait()
        plsc.subcore_barrier()

        @pl.when(sid < NS - 1)
        def _():
            def w(p_, _):
                pltpu.sync_copy(agg_sp.at[pl.ds(off + ZP * p_, ZP)], zbuf)
                pltpu.sync_copy(zbuf, out_j.at[pl.ds(off + ZP * p_, ZP)])
                return 0
            lax.fori_loop(0, STRIPE // ZP, w, 0)

        @pl.when(sid == NS - 1)
        def _():
            def w(p_, _):
                pltpu.sync_copy(agg_sp.at[pl.ds(off + ZP * p_, ZP)], zbuf)
                pltpu.sync_copy(zbuf, out_j.at[pl.ds(off + ZP * p_, ZP)])
                return 0
            lax.fori_loop(0, STRIPE_LAST // ZP, w, 0)
            rem = STRIPE_LAST - (STRIPE_LAST // ZP) * ZP
            pltpu.sync_copy(agg_sp.at[pl.ds(off + STRIPE_LAST - rem, rem)],
                            zbuf.at[pl.ds(0, rem)])
            pltpu.sync_copy(zbuf.at[pl.ds(0, rem)],
                            out_j.at[pl.ds(off + STRIPE_LAST - rem, rem)])

        plsc.subcore_barrier()

    npc = len(hs_list) // 2

    @pl.when(cid == 0)
    def _():
        for t in range(npc):
            process(hs_list[t], out_list[t])

    @pl.when(cid == 1)
    def _():
        for t in range(npc):
            process(hs_list[npc + t], out_list[npc + t])


def _agg_scratch():
    return [
        pltpu.VMEM_SHARED((NROW, CW), f32),
        pltpu.VMEM((2, PIECE, K), jnp.int32),
        pltpu.VMEM((2, PIECE, K), jnp.int32),
        pltpu.VMEM((4, K, CW), f32),
        pltpu.VMEM((ZP, CW), f32),
        pltpu.SemaphoreType.DMA((4,)),
        pltpu.SemaphoreType.DMA((4,)),
        pltpu.SemaphoreType.DMA,
    ]


@functools.partial(
    pl.kernel,
    out_type=[jax.ShapeDtypeStruct((N, CW), f32) for _ in range(2)],
    mesh=_mesh,
    compiler_params=_sc_params,
    scratch_types=_agg_scratch(),
)
def _sc_agg2(h0, h1, src_hbm, dst_hbm, o0, o1, *scr):
    _agg_impl([h0, h1], src_hbm, dst_hbm, [o0, o1], *scr)


@functools.partial(
    pl.kernel,
    out_type=[jax.ShapeDtypeStruct((N, CW), f32) for _ in range(4)],
    mesh=_mesh,
    compiler_params=_sc_params,
    scratch_types=_agg_scratch(),
)
def _sc_agg4(h0, h1, h2, h3, src_hbm, dst_hbm, o0, o1, o2, o3, *scr):
    _agg_impl([h0, h1, h2, h3], src_hbm, dst_hbm, [o0, o1, o2, o3], *scr)


RB = 2000
NRB = N // RB
EPS = 1e-5


def _deg_scale(degp_ref):
    deg = jnp.sum(degp_ref[...], axis=1, keepdims=True)
    return lax.rsqrt(deg)


def _tca_body(x_ref, degp_ref, o0, o1):
    d = _deg_scale(degp_ref)
    xs = x_ref[...] * d
    o0[...] = xs[:, :CW]
    o1[...] = xs[:, CW:]


def _tc_a(x, degp):
    return pl.pallas_call(
        _tca_body,
        grid=(NRB,),
        in_specs=[
            pl.BlockSpec((RB, IN), lambda i: (i, 0)),
            pl.BlockSpec((RB, 2), lambda i: (i, 0)),
        ],
        out_specs=[pl.BlockSpec((RB, CW), lambda i: (i, 0))] * 2,
        out_shape=[jax.ShapeDtypeStruct((N, CW), f32)] * 2,
    )(x, degp)


def _relu1(a0, a1, degp_ref, w1_ref, b1_ref):
    d = _deg_scale(degp_ref)
    ar = jnp.concatenate([a0[...], a1[...]], axis=1)
    h = jnp.dot(ar, w1_ref[...], preferred_element_type=f32)
    return jnp.maximum(h * d + b1_ref[...], 0.0), d


def _tcb1_body(a0, a1, degp_ref, w1_ref, b1_ref, stats_ref):
    r, _ = _relu1(a0, a1, degp_ref, w1_ref, b1_ref)
    stats_ref[0, 0, :] = jnp.sum(r, axis=0)
    stats_ref[0, 1, :] = jnp.sum(r * r, axis=0)


def _tc_b1(aggs, degp, W1, b1):
    return pl.pallas_call(
        _tcb1_body,
        grid=(NRB,),
        in_specs=[pl.BlockSpec((RB, CW), lambda i: (i, 0))] * 2 + [
            pl.BlockSpec((RB, 2), lambda i: (i, 0)),
            pl.BlockSpec((IN, H), lambda i: (0, 0)),
            pl.BlockSpec((H,), lambda i: (0,)),
        ],
        out_specs=pl.BlockSpec((1, 2, H), lambda i: (i, 0, 0)),
        out_shape=jax.ShapeDtypeStruct((NRB, 2, H), f32),
    )(*aggs, degp, W1, b1)


def _tcb2_body(a0, a1, degp_ref, w1_ref, b1_ref, stats_ref, g_ref, be_ref,
               o0, o1, o2, o3):
    r, d = _relu1(a0, a1, degp_ref, w1_ref, b1_ref)
    s = stats_ref[...]
    mean = jnp.sum(s[:, 0], axis=0) * (1.0 / N)
    var = jnp.sum(s[:, 1], axis=0) * (1.0 / N) - mean * mean
    inv = lax.rsqrt(var + EPS)
    y = (r - mean) * (inv * g_ref[...]) + be_ref[...]
    xs2 = y * d
    for j, o in enumerate((o0, o1, o2, o3)):
        o[...] = xs2[:, CW * j:CW * (j + 1)]


def _tc_b2(aggs, degp, W1, b1, stats, gamma, beta):
    return pl.pallas_call(
        _tcb2_body,
        grid=(NRB,),
        in_specs=[pl.BlockSpec((RB, CW), lambda i: (i, 0))] * 2 + [
            pl.BlockSpec((RB, 2), lambda i: (i, 0)),
            pl.BlockSpec((IN, H), lambda i: (0, 0)),
            pl.BlockSpec((H,), lambda i: (0,)),
            pl.BlockSpec((NRB, 2, H), lambda i: (0, 0, 0)),
            pl.BlockSpec((H,), lambda i: (0,)),
            pl.BlockSpec((H,), lambda i: (0,)),
        ],
        out_specs=[pl.BlockSpec((RB, CW), lambda i: (i, 0))] * 4,
        out_shape=[jax.ShapeDtypeStruct((N, CW), f32)] * 4,
    )(*aggs, degp, W1, b1, stats, gamma, beta)


def _tcc_body(a0, a1, a2, a3, degp_ref, w2_ref, b2_ref, out_ref):
    d = _deg_scale(degp_ref)
    ag = jnp.concatenate([a0[...], a1[...], a2[...], a3[...]], axis=1)
    h = jnp.dot(ag, w2_ref[...], preferred_element_type=f32)
    out_ref[...] = jnp.maximum(h * d + b2_ref[...], 0.0)


def _tc_c(aggs, degp, W2, b2):
    return pl.pallas_call(
        _tcc_body,
        grid=(NRB,),
        in_specs=[pl.BlockSpec((RB, CW), lambda i: (i, 0))] * 4 + [
            pl.BlockSpec((RB, 2), lambda i: (i, 0)),
            pl.BlockSpec((H, H), lambda i: (0, 0)),
            pl.BlockSpec((H,), lambda i: (0,)),
        ],
        out_specs=pl.BlockSpec((RB, H), lambda i: (i, 0)),
        out_shape=jax.ShapeDtypeStruct((N, H), f32),
    )(*aggs, degp, W2, b2)


def kernel(x, edge_index, W1, b1, gamma, beta, W2, b2):
    src = edge_index[0]
    dst = edge_index[1]
    loop = jnp.arange(N, dtype=jnp.int32)
    npad = EPAD - EE
    src_p = jnp.concatenate([src, loop, jnp.zeros((npad,), jnp.int32)])
    dst_p = jnp.concatenate([dst, loop, jnp.full((npad,), DUMP, jnp.int32)])
    src_ag = src_p.reshape(NS, NB_AG, K)
    dst_ag = dst_p.reshape(NS, NB_AG, K)
    dst_dg = dst_p.reshape(NC * NS, NB_DG, K)

    degp = _sc_deg(dst_dg).reshape(NC, N).T
    xs1 = _tc_a(x, degp)
    agg1 = _sc_agg2(*xs1, src_ag, dst_ag)
    stats = _tc_b1(agg1, degp, W1, b1)
    xs2 = _tc_b2(agg1, degp, W1, b1, stats, gamma, beta)
    agg2 = _sc_agg4(*xs2, src_ag, dst_ag)
    return _tc_c(agg2, degp, W2, b2)
